# baseline (speedup 1.0000x reference)
import functools

import jax
import jax.numpy as jnp
from jax import lax
from jax.experimental import pallas as pl
from jax.experimental.pallas import tpu as pltpu
from jax.experimental.pallas import tpu_sc as plsc

D = 16
B = 4096
NK = 50
NC = 2
NS = 16
L = 16
NW = NC * NS
BPW = B // NW
G = BPW // L

_LN2 = 0.6931471805599453


def _rsqrt(y):
    i = plsc.bitcast(y, jnp.int32)
    i = 0x5F3759DF - (i >> 1)
    r = plsc.bitcast(i, jnp.float32)
    r = r * (1.5 - 0.5 * y * r * r)
    r = r * (1.5 - 0.5 * y * r * r)
    return r


def _log(x):
    i = plsc.bitcast(x, jnp.int32)
    k = ((i >> 23) - 127).astype(jnp.float32)
    m = plsc.bitcast((i & 0x007FFFFF) | 0x3F800000, jnp.float32)
    z = (m - 1.0) / (m + 1.0)
    z2 = z * z
    p = 2.0 * z * (1.0 + z2 * (1.0 / 3.0 + z2 * (1.0 / 5.0 + z2 * (1.0 / 7.0 + z2 * (1.0 / 9.0)))))
    return k * _LN2 + p


_TBC = 1024
_TGRID = -(-(_N_ROWS := 1000001) // _TBC)
_ROWS128 = _TGRID * _TBC // 8


def _tc_detile_body(in_ref, out_ref):
    x = in_ref[...]
    e = x.reshape(16, _TBC // 8, 8)
    t = jnp.transpose(e, (1, 2, 0))
    out_ref[...] = t.reshape(_TBC // 8, 128)


def _tc_detile(table_t):
    return pl.pallas_call(
        _tc_detile_body,
        grid=(_TGRID,),
        in_specs=[pl.BlockSpec((16, _TBC), lambda i: (0, i))],
        out_specs=pl.BlockSpec((_TBC // 8, 128), lambda i: (i, 0)),
        out_shape=jax.ShapeDtypeStruct((_ROWS128, 128), jnp.float32),
    )(table_t)


def _sc_lorentz(table, i_arr, ks):
    mesh = plsc.VectorSubcoreMesh(core_axis_name="c", subcore_axis_name="s")

    nrows = table.shape[0]

    @functools.partial(
        pl.kernel,
        out_type=jax.ShapeDtypeStruct((B,), jnp.float32),
        mesh=mesh,
        compiler_params=pltpu.CompilerParams(
            needs_layout_passes=False, use_tc_tiling_on_sc=False),
        scratch_types=[
            pltpu.VMEM((BPW, NK), jnp.int32),
            pltpu.VMEM((NK, BPW), jnp.int32),
            pltpu.VMEM((BPW,), jnp.int32),
            pltpu.VMEM((BPW, D), jnp.float32),
            pltpu.VMEM((NK * BPW, D), jnp.float32),
            pltpu.VMEM((BPW,), jnp.float32),
            pltpu.SemaphoreType.DMA,
            pltpu.SemaphoreType.DMA,
        ],
    )
    def k(table_hbm, i_hbm, ks_hbm, out_hbm,
          ks_blk, ks_idx, i_idx, ui_rows, uk_rows, loss_v, sem_ui, sem_uk):
        wid = lax.axis_index("s") * NC + lax.axis_index("c")
        base = wid * BPW

        pltpu.sync_copy(i_hbm.at[pl.ds(base, BPW)], i_idx)
        pltpu.sync_copy(ks_hbm.at[pl.ds(base, BPW)], ks_blk)

        ui_cp = pltpu.async_copy(table_hbm.at[i_idx], ui_rows, sem_ui)

        iota = lax.iota(jnp.int32, L)
        dsplat = [jnp.full((L,), d, jnp.int32) for d in range(D)]
        nsplat = [jnp.full((L,), n, jnp.int32) for n in range(NK)]

        uk_cps = []
        for n in range(NK):
            for g in range(G):
                v = plsc.load_gather(ks_blk, [iota + g * L, nsplat[n]])
                ks_idx[n, pl.ds(g * L, L)] = v
            uk_cps.append(pltpu.async_copy(
                table_hbm.at[ks_idx.at[n]],
                uk_rows.at[pl.ds(n * BPW, BPW)], sem_uk))

        ui_cp.wait()
        for cp in uk_cps:
            cp.wait()

        for g in range(G):
            gidx = iota + (g * L)
            uiT = []
            for d in range(D):
                v = plsc.load_gather(ui_rows, [gidx, dsplat[d]])
                uiT.append(-v if d == 0 else v)

            def pair_e(n):
                ridx = iota + (n * BPW + g * L)
                acc = uiT[0] * plsc.load_gather(uk_rows, [ridx, dsplat[0]])
                for d in range(1, D):
                    acc = acc + uiT[d] * plsc.load_gather(uk_rows, [ridx, dsplat[d]])
                dd = jnp.maximum(-acc, 1.0)
                y = (dd - 1.0) * (dd + 1.0)
                return dd - y * _rsqrt(y)

            e0 = pair_e(0)

            def body(n, accS):
                return accS + pair_e(n)

            accS = lax.fori_loop(1, NK, body, e0)
            lg = _log((accS + 1e-6) / e0)
            loss_v[pl.ds(g * L, L)] = lg

        pltpu.sync_copy(loss_v, out_hbm.at[pl.ds(base, BPW)])

    return k(table, i_arr, ks)


def kernel(table, I, Ks):
    tbl = _tc_detile(table.astype(jnp.float32).T).reshape(-1, D)
    return _sc_lorentz(tbl, I.astype(jnp.int32), Ks.astype(jnp.int32))


# speedup vs baseline: 6.9525x; 6.9525x over previous
import functools

import jax
import jax.numpy as jnp
from jax import lax
from jax.experimental import pallas as pl
from jax.experimental.pallas import tpu as pltpu
from jax.experimental.pallas import tpu_sc as plsc

D = 16
B = 4096
NK = 50
NC = 2
NS = 16
L = 16
NW = NC * NS
BPW = B // NW
G = BPW // L

_LN2 = 0.6931471805599453


def _rsqrt(y):
    i = plsc.bitcast(y, jnp.int32)
    i = 0x5F3759DF - (i >> 1)
    r = plsc.bitcast(i, jnp.float32)
    r = r * (1.5 - 0.5 * y * r * r)
    r = r * (1.5 - 0.5 * y * r * r)
    return r


def _log(x):
    i = plsc.bitcast(x, jnp.int32)
    k = ((i >> 23) - 127).astype(jnp.float32)
    m = plsc.bitcast((i & 0x007FFFFF) | 0x3F800000, jnp.float32)
    z = (m - 1.0) / (m + 1.0)
    z2 = z * z
    p = 2.0 * z * (1.0 + z2 * (1.0 / 3.0 + z2 * (1.0 / 5.0 + z2 * (1.0 / 7.0 + z2 * (1.0 / 9.0)))))
    return k * _LN2 + p


_TCB = 1024
_STEP = 8 * _TCB
_TGRID = -(-1000001 // _STEP)
_ROWS128 = _TGRID * _STEP // 8


def _tc_detile_body(x_ref, out_ref):
    x = x_ref[...]
    for q in range(8):
        x8 = jnp.concatenate(
            [x[:, j * _TCB + q * 128:j * _TCB + (q + 1) * 128]
             for j in range(8)], axis=0)
        out_ref[q * 128:(q + 1) * 128, :] = x8.T


def _tc_detile(table_t):
    return pl.pallas_call(
        _tc_detile_body,
        grid=(_TGRID,),
        in_specs=[pl.BlockSpec((16, _STEP), lambda i: (0, i))],
        out_specs=pl.BlockSpec((_STEP // 8, 128), lambda i: (i, 0)),
        out_shape=jax.ShapeDtypeStruct((_ROWS128, 128), jnp.float32),
    )(table_t)


def _perm_rows(v):
    return (v & -8192) | ((v & 1023) << 3) | ((v >> 10) & 7)


def _sc_lorentz(table, i_arr, ks):
    mesh = plsc.VectorSubcoreMesh(core_axis_name="c", subcore_axis_name="s")

    nrows = table.shape[0]

    @functools.partial(
        pl.kernel,
        out_type=jax.ShapeDtypeStruct((B,), jnp.float32),
        mesh=mesh,
        compiler_params=pltpu.CompilerParams(
            needs_layout_passes=False, use_tc_tiling_on_sc=False),
        scratch_types=[
            pltpu.VMEM((BPW, NK), jnp.int32),
            pltpu.VMEM((NK, BPW), jnp.int32),
            pltpu.VMEM((BPW,), jnp.int32),
            pltpu.VMEM((BPW, D), jnp.float32),
            pltpu.VMEM((NK * BPW, D), jnp.float32),
            pltpu.VMEM((BPW,), jnp.float32),
            pltpu.SemaphoreType.DMA,
            pltpu.SemaphoreType.DMA,
        ],
    )
    def k(table_hbm, i_hbm, ks_hbm, out_hbm,
          ks_blk, ks_idx, i_idx, ui_rows, uk_rows, loss_v, sem_ui, sem_uk):
        wid = lax.axis_index("s") * NC + lax.axis_index("c")
        base = wid * BPW

        pltpu.sync_copy(i_hbm.at[pl.ds(base, BPW)], i_idx)
        pltpu.sync_copy(ks_hbm.at[pl.ds(base, BPW)], ks_blk)

        iota = lax.iota(jnp.int32, L)
        dsplat = [jnp.full((L,), d, jnp.int32) for d in range(D)]
        nsplat = [jnp.full((L,), n, jnp.int32) for n in range(NK)]

        for g in range(G):
            i_idx[pl.ds(g * L, L)] = _perm_rows(i_idx[pl.ds(g * L, L)])
        ui_cp = pltpu.async_copy(table_hbm.at[i_idx], ui_rows, sem_ui)

        uk_cps = []
        for n in range(NK):
            for g in range(G):
                v = plsc.load_gather(ks_blk, [iota + g * L, nsplat[n]])
                ks_idx[n, pl.ds(g * L, L)] = _perm_rows(v)
            uk_cps.append(pltpu.async_copy(
                table_hbm.at[ks_idx.at[n]],
                uk_rows.at[pl.ds(n * BPW, BPW)], sem_uk))

        ui_cp.wait()
        for cp in uk_cps:
            cp.wait()

        for g in range(G):
            gidx = iota + (g * L)
            uiT = []
            for d in range(D):
                v = plsc.load_gather(ui_rows, [gidx, dsplat[d]])
                uiT.append(-v if d == 0 else v)

            def pair_e(n):
                ridx = iota + (n * BPW + g * L)
                acc = uiT[0] * plsc.load_gather(uk_rows, [ridx, dsplat[0]])
                for d in range(1, D):
                    acc = acc + uiT[d] * plsc.load_gather(uk_rows, [ridx, dsplat[d]])
                dd = jnp.maximum(-acc, 1.0)
                y = (dd - 1.0) * (dd + 1.0)
                return dd - y * _rsqrt(y)

            e0 = pair_e(0)

            def body(n, accS):
                return accS + pair_e(n)

            accS = lax.fori_loop(1, NK, body, e0)
            lg = _log((accS + 1e-6) / e0)
            loss_v[pl.ds(g * L, L)] = lg

        pltpu.sync_copy(loss_v, out_hbm.at[pl.ds(base, BPW)])

    return k(table, i_arr, ks)


def kernel(table, I, Ks):
    tbl = _tc_detile(table.astype(jnp.float32).T).reshape(-1, D)
    return _sc_lorentz(tbl, I.astype(jnp.int32), Ks.astype(jnp.int32))


# speedup vs baseline: 8.5143x; 1.2246x over previous
import functools

import jax
import jax.numpy as jnp
from jax import lax
from jax.experimental import pallas as pl
from jax.experimental.pallas import tpu as pltpu
from jax.experimental.pallas import tpu_sc as plsc

D = 16
B = 4096
NK = 50
NC = 2
NS = 16
L = 16
NW = NC * NS
BPW = B // NW
G = BPW // L

_LN2 = 0.6931471805599453


def _rsqrt(y):
    i = plsc.bitcast(y, jnp.int32)
    i = 0x5F3759DF - (i >> 1)
    r = plsc.bitcast(i, jnp.float32)
    r = r * (1.5 - 0.5 * y * r * r)
    r = r * (1.5 - 0.5 * y * r * r)
    return r


def _log(x):
    i = plsc.bitcast(x, jnp.int32)
    k = ((i >> 23) - 127).astype(jnp.float32)
    m = plsc.bitcast((i & 0x007FFFFF) | 0x3F800000, jnp.float32)
    z = (m - 1.0) / (m + 1.0)
    z2 = z * z
    p = 2.0 * z * (1.0 + z2 * (1.0 / 3.0 + z2 * (1.0 / 5.0 + z2 * (1.0 / 7.0 + z2 * (1.0 / 9.0)))))
    return k * _LN2 + p


_TCB = 2048
_STEP = 8 * _TCB
_TGRID = -(-1000001 // _STEP)
_ROWS128 = _TGRID * _STEP // 8


def _tc_detile_body(x_ref, out_ref):
    x = x_ref[...]
    for q in range(_TCB // 128):
        x8 = jnp.concatenate(
            [x[:, j * _TCB + q * 128:j * _TCB + (q + 1) * 128]
             for j in range(8)], axis=0)
        out_ref[q * 128:(q + 1) * 128, :] = x8.T


def _tc_detile(table_t):
    return pl.pallas_call(
        _tc_detile_body,
        grid=(_TGRID,),
        in_specs=[pl.BlockSpec((16, _STEP), lambda i: (0, i))],
        out_specs=pl.BlockSpec((_STEP // 8, 128), lambda i: (i, 0)),
        out_shape=jax.ShapeDtypeStruct((_ROWS128, 128), jnp.float32),
    )(table_t)


_TCB_BITS = _TCB.bit_length() - 1


def _perm_rows(v):
    return (v & -_STEP) | ((v & (_TCB - 1)) << 3) | ((v >> _TCB_BITS) & 7)


def _sc_lorentz(table, i_arr, ks):
    mesh = plsc.VectorSubcoreMesh(core_axis_name="c", subcore_axis_name="s")

    nrows = table.shape[0]

    @functools.partial(
        pl.kernel,
        out_type=jax.ShapeDtypeStruct((B,), jnp.float32),
        mesh=mesh,
        compiler_params=pltpu.CompilerParams(
            needs_layout_passes=False, use_tc_tiling_on_sc=False),
        scratch_types=[
            pltpu.VMEM((BPW, NK), jnp.int32),
            pltpu.VMEM((NK, BPW), jnp.int32),
            pltpu.VMEM((BPW,), jnp.int32),
            pltpu.VMEM((BPW, D), jnp.float32),
            pltpu.VMEM((NK * BPW, D), jnp.float32),
            pltpu.VMEM((BPW,), jnp.float32),
            pltpu.VMEM((G, L), jnp.float32),
            pltpu.VMEM((G, L), jnp.float32),
            pltpu.SemaphoreType.DMA,
            pltpu.SemaphoreType.DMA,
            pltpu.SemaphoreType.DMA,
        ],
    )
    def k(table_hbm, i_hbm, ks_hbm, out_hbm,
          ks_blk, ks_idx, i_idx, ui_rows, uk_rows, loss_v, accS_v, e0_v,
          sem_ui, sem_uk0, sem_uk1):
        wid = lax.axis_index("s") * NC + lax.axis_index("c")
        base = wid * BPW

        pltpu.sync_copy(i_hbm.at[pl.ds(base, BPW)], i_idx)
        pltpu.sync_copy(ks_hbm.at[pl.ds(base, BPW)], ks_blk)

        iota = lax.iota(jnp.int32, L)
        dsplat = [jnp.full((L,), d, jnp.int32) for d in range(D)]
        nsplat = [jnp.full((L,), n, jnp.int32) for n in range(NK)]

        for g in range(G):
            i_idx[pl.ds(g * L, L)] = _perm_rows(i_idx[pl.ds(g * L, L)])
        ui_cp = pltpu.async_copy(table_hbm.at[i_idx], ui_rows, sem_ui)

        HALF = NK // 2
        uk_cps = []
        for n in range(NK):
            for g in range(G):
                v = plsc.load_gather(ks_blk, [iota + g * L, nsplat[n]])
                ks_idx[n, pl.ds(g * L, L)] = _perm_rows(v)
            uk_cps.append(pltpu.async_copy(
                table_hbm.at[ks_idx.at[n]],
                uk_rows.at[pl.ds(n * BPW, BPW)],
                sem_uk0 if n < HALF else sem_uk1))

        ui_cp.wait()

        def pair_e(g, n, uiT):
            ridx = iota + (n * BPW + g * L)
            acc = uiT[0] * plsc.load_gather(uk_rows, [ridx, dsplat[0]])
            for d in range(1, D):
                acc = acc + uiT[d] * plsc.load_gather(uk_rows, [ridx, dsplat[d]])
            dd = jnp.maximum(-acc, 1.0)
            y = (dd - 1.0) * (dd + 1.0)
            return dd - y * _rsqrt(y)

        def load_uiT(g):
            gidx = iota + (g * L)
            uiT = []
            for d in range(D):
                v = plsc.load_gather(ui_rows, [gidx, dsplat[d]])
                uiT.append(-v if d == 0 else v)
            return uiT

        for cp in uk_cps[:HALF]:
            cp.wait()
        for g in range(G):
            uiT = load_uiT(g)
            e0 = pair_e(g, 0, uiT)

            def body(n, accS):
                return accS + pair_e(g, n, uiT)

            accS_v[g] = lax.fori_loop(1, HALF, body, e0)
            e0_v[g] = e0

        for cp in uk_cps[HALF:]:
            cp.wait()
        for g in range(G):
            uiT = load_uiT(g)

            def body(n, accS):
                return accS + pair_e(g, n, uiT)

            accS = lax.fori_loop(HALF, NK, body, accS_v[g])
            lg = _log((accS + 1e-6) / e0_v[g])
            loss_v[pl.ds(g * L, L)] = lg

        pltpu.sync_copy(loss_v, out_hbm.at[pl.ds(base, BPW)])

    return k(table, i_arr, ks)


def kernel(table, I, Ks):
    tbl = _tc_detile(table.astype(jnp.float32).T).reshape(-1, D)
    return _sc_lorentz(tbl, I.astype(jnp.int32), Ks.astype(jnp.int32))
